# Initial kernel scaffold; baseline (speedup 1.0000x reference)
import jax
import jax.numpy as jnp
from jax.experimental import pallas as pl


def kernel(hidden_states, gate_w, up_w, down_w, expert_weight, ln_gamma, ln_beta):
    raise NotImplementedError("write your pallas kernel here")


# speedup vs baseline: 1.1433x; 1.1433x over previous
import functools

import jax
import jax.numpy as jnp
from jax.experimental import pallas as pl
from jax.experimental.pallas import tpu as pltpu


def _fused_body(hs_ref, gate_ref, up_ref, down_ref, ew_ref, g_ref, b_ref,
                out_ref, scale_ref, *, ksteps, kh):
    k = pl.program_id(1)
    hs = hs_ref[...]

    @pl.when(k == 0)
    def _init():
        logits = jax.lax.dot_general(
            hs, gate_ref[...], (((1,), (1,)), ((), ())),
            preferred_element_type=jnp.float32)
        m1 = jnp.max(logits, axis=-1, keepdims=True)
        is_max = logits >= m1
        cnt = jnp.sum(is_max.astype(jnp.float32), axis=-1, keepdims=True)
        rest = jnp.where(is_max, -jnp.inf, logits)
        m2 = jnp.max(rest, axis=-1, keepdims=True)
        scale_ref[...] = jnp.where(cnt >= 2.0, 2.0 * m1, m1 + m2)
        out_ref[...] = jnp.zeros_like(out_ref)

    u = jax.lax.dot_general(
        hs, up_ref[...], (((1,), (1,)), ((), ())),
        preferred_element_type=jnp.float32)
    a = jnp.maximum(u, 0.0)
    a = (a * a).astype(jnp.bfloat16)
    part = jax.lax.dot_general(
        a, down_ref[...], (((1,), (1,)), ((), ())),
        preferred_element_type=jnp.float32)
    hs_k = hs_ref[:, pl.ds(k * kh, kh)]
    moe = jax.lax.dot_general(
        hs_k, ew_ref[...], (((1,), (0,)), ((), ())),
        preferred_element_type=jnp.float32)
    out_ref[...] += part + moe * scale_ref[...]

    @pl.when(k == ksteps - 1)
    def _finish():
        acc = out_ref[...]
        mu = jnp.mean(acc, axis=-1, keepdims=True)
        var = jnp.mean((acc - mu) ** 2, axis=-1, keepdims=True)
        out_ref[...] = ((acc - mu) * jax.lax.rsqrt(var + 1e-5)
                        * g_ref[...] + b_ref[...])


def kernel(hidden_states, gate_w, up_w, down_w, expert_weight, ln_gamma,
           ln_beta):
    tokens, hidden = hidden_states.shape
    inter = up_w.shape[0]
    nexp = gate_w.shape[0]

    ksteps = 8
    ki = inter // ksteps
    kh = hidden // ksteps
    tm = 1024 if tokens % 1024 == 0 else tokens

    hs = hidden_states.astype(jnp.bfloat16)
    gate = gate_w.astype(jnp.bfloat16)
    up = up_w.astype(jnp.bfloat16)
    down = down_w.astype(jnp.bfloat16)
    ew = expert_weight.astype(jnp.bfloat16)
    gamma = ln_gamma.reshape(1, hidden)
    beta = ln_beta.reshape(1, hidden)

    grid = (tokens // tm, ksteps)
    out = pl.pallas_call(
        functools.partial(_fused_body, ksteps=ksteps, kh=kh),
        grid=grid,
        in_specs=[
            pl.BlockSpec((tm, hidden), lambda i, k: (i, 0)),
            pl.BlockSpec((nexp, hidden), lambda i, k: (0, 0)),
            pl.BlockSpec((ki, hidden), lambda i, k: (k, 0)),
            pl.BlockSpec((hidden, ki), lambda i, k: (0, k)),
            pl.BlockSpec((kh, hidden), lambda i, k: (k, 0)),
            pl.BlockSpec((1, hidden), lambda i, k: (0, 0)),
            pl.BlockSpec((1, hidden), lambda i, k: (0, 0)),
        ],
        out_specs=pl.BlockSpec((tm, hidden), lambda i, k: (i, 0)),
        out_shape=jax.ShapeDtypeStruct((tokens, hidden), jnp.float32),
        scratch_shapes=[pltpu.VMEM((tm, 1), jnp.float32)],
        compiler_params=pltpu.CompilerParams(
            dimension_semantics=("parallel", "arbitrary")),
    )(hs, gate, up, down, ew, gamma, beta)
    return out
